# baseline (speedup 1.0000x reference)
import jax
import jax.numpy as jnp
from jax.experimental import pallas as pl
from jax.experimental.pallas import tpu as pltpu

_MAX_CTX = 32768
_D = 1024
_NBUF = 16
_CHUNK_ROWS = 512
_NCHUNKS = _MAX_CTX // _CHUNK_ROWS
assert _NCHUNKS % _NBUF == 0


def _copy_kernel(src_ref, dst_ref, buf, rsem, wsem):
    ngroups = _NCHUNKS // _NBUF
    for g in range(ngroups):
        for b in range(_NBUF):
            c = g * _NBUF + b
            pltpu.make_async_copy(
                src_ref.at[pl.ds(c * _CHUNK_ROWS, _CHUNK_ROWS), :], buf.at[b], rsem.at[b]
            ).start()
        for b in range(_NBUF):
            c = g * _NBUF + b
            pltpu.make_async_copy(
                src_ref.at[pl.ds(c * _CHUNK_ROWS, _CHUNK_ROWS), :], buf.at[b], rsem.at[b]
            ).wait()
        for b in range(_NBUF):
            c = g * _NBUF + b
            pltpu.make_async_copy(
                buf.at[b], dst_ref.at[pl.ds(c * _CHUNK_ROWS, _CHUNK_ROWS), :], wsem.at[b]
            ).start()
        for b in range(_NBUF):
            c = g * _NBUF + b
            pltpu.make_async_copy(
                buf.at[b], dst_ref.at[pl.ds(c * _CHUNK_ROWS, _CHUNK_ROWS), :], wsem.at[b]
            ).wait()


def kernel(h, mem):
    b, l, d = h.shape
    assert b * l == _MAX_CTX and d == _D
    flat = h.reshape(b * l, d)
    new_mem = pl.pallas_call(
        _copy_kernel,
        in_specs=[pl.BlockSpec(memory_space=pl.ANY)],
        out_specs=pl.BlockSpec(memory_space=pl.ANY),
        out_shape=jax.ShapeDtypeStruct((b * l, d), h.dtype),
        scratch_shapes=[
            pltpu.VMEM((_NBUF, _CHUNK_ROWS, _D), h.dtype),
            pltpu.SemaphoreType.DMA((_NBUF,)),
            pltpu.SemaphoreType.DMA((_NBUF,)),
        ],
        compiler_params=pltpu.CompilerParams(
            disable_bounds_checks=True,
            disable_semaphore_checks=True,
            skip_device_barrier=True,
        ),
    )(flat)
    return (h, new_mem)


# speedup vs baseline: 1.3846x; 1.3846x over previous
import jax
import jax.numpy as jnp
from jax.experimental import pallas as pl
from jax.experimental.pallas import tpu as pltpu

_MAX_CTX = 32768
_D = 1024
_NBUF = 4
_CHUNK = 2048
_NCHUNKS = _MAX_CTX // _CHUNK
assert _NCHUNKS % _NBUF == 0


def _copy_kernel(src_ref, out_h_ref, out_mem_ref, buf, rsem, w1sem, w2sem):
    rows_per_batch = out_h_ref.shape[1]
    chunks_per_batch = rows_per_batch // _CHUNK

    def h_slot(c):
        return (c // chunks_per_batch, pl.ds((c % chunks_per_batch) * _CHUNK, _CHUNK))

    ngroups = _NCHUNKS // _NBUF
    for g in range(ngroups):
        for b in range(_NBUF):
            c = g * _NBUF + b
            if g > 0:
                pc = c - _NBUF
                pltpu.make_async_copy(
                    buf.at[b], out_mem_ref.at[pl.ds(pc * _CHUNK, _CHUNK), :], w1sem.at[b]
                ).wait()
                bi, rs = h_slot(pc)
                pltpu.make_async_copy(
                    buf.at[b], out_h_ref.at[bi, rs, :], w2sem.at[b]
                ).wait()
            pltpu.make_async_copy(
                src_ref.at[pl.ds(c * _CHUNK, _CHUNK), :], buf.at[b], rsem.at[b]
            ).start()
        for b in range(_NBUF):
            c = g * _NBUF + b
            pltpu.make_async_copy(
                src_ref.at[pl.ds(c * _CHUNK, _CHUNK), :], buf.at[b], rsem.at[b]
            ).wait()
            pltpu.make_async_copy(
                buf.at[b], out_mem_ref.at[pl.ds(c * _CHUNK, _CHUNK), :], w1sem.at[b]
            ).start()
            bi, rs = h_slot(c)
            pltpu.make_async_copy(
                buf.at[b], out_h_ref.at[bi, rs, :], w2sem.at[b]
            ).start()
    g = ngroups - 1
    for b in range(_NBUF):
        c = g * _NBUF + b
        pltpu.make_async_copy(
            buf.at[b], out_mem_ref.at[pl.ds(c * _CHUNK, _CHUNK), :], w1sem.at[b]
        ).wait()
        bi, rs = h_slot(c)
        pltpu.make_async_copy(
            buf.at[b], out_h_ref.at[bi, rs, :], w2sem.at[b]
        ).wait()


def kernel(h, mem):
    b, l, d = h.shape
    assert b * l == _MAX_CTX and d == _D
    flat = h.reshape(b * l, d)
    out_h, new_mem = pl.pallas_call(
        _copy_kernel,
        in_specs=[pl.BlockSpec(memory_space=pl.ANY)],
        out_specs=[
            pl.BlockSpec(memory_space=pl.ANY),
            pl.BlockSpec(memory_space=pl.ANY),
        ],
        out_shape=[
            jax.ShapeDtypeStruct((b, l, d), h.dtype),
            jax.ShapeDtypeStruct((b * l, d), h.dtype),
        ],
        scratch_shapes=[
            pltpu.VMEM((_NBUF, _CHUNK, _D), h.dtype),
            pltpu.SemaphoreType.DMA((_NBUF,)),
            pltpu.SemaphoreType.DMA((_NBUF,)),
            pltpu.SemaphoreType.DMA((_NBUF,)),
        ],
        compiler_params=pltpu.CompilerParams(
            disable_bounds_checks=True,
            disable_semaphore_checks=True,
            skip_device_barrier=True,
        ),
    )(flat)
    return (out_h, new_mem)
